# baseline (speedup 1.0000x reference)
import functools

import jax
import jax.numpy as jnp
from jax import lax
from jax.experimental import pallas as pl
from jax.experimental.pallas import tpu as pltpu
from jax.experimental.pallas import tpu_sc as plsc

NC = 2
NS = 16
L = 16
NW = NC * NS


def _sc_spmm(x, srcp, dstp, wp, zeros, n, d, nb):
    mesh = plsc.VectorSubcoreMesh(
        core_axis_name="c", subcore_axis_name="s", num_cores=NC, num_subcores=NS
    )
    rpt = 8 * (-(-n // (NS * 8)))
    last = n - (NS - 1) * rpt

    @functools.partial(
        pl.kernel,
        out_type=jax.ShapeDtypeStruct((NC, n, d), jnp.float32),
        mesh=mesh,
        compiler_params=pltpu.CompilerParams(needs_layout_passes=False),
        scratch_types=[
            pltpu.VMEM((3, 128), jnp.int32),
            pltpu.VMEM((3, 128), jnp.int32),
            pltpu.VMEM((3, 128), jnp.float32),
            pltpu.VMEM((128, d), jnp.float32),
            pltpu.VMEM((128, d), jnp.float32),
            pltpu.VMEM((128, d), jnp.float32),
            pltpu.VMEM_SHARED((n, d), jnp.float32),
            pltpu.SemaphoreType.DMA,
            pltpu.SemaphoreType.DMA,
            pltpu.SemaphoreType.DMA,
            pltpu.SemaphoreType.DMA,
            pltpu.SemaphoreType.DMA,
            pltpu.SemaphoreType.DMA,
            pltpu.SemaphoreType.DMA,
            pltpu.SemaphoreType.DMA,
            pltpu.SemaphoreType.DMA,
            pltpu.SemaphoreType.DMA,
            pltpu.SemaphoreType.DMA,
            pltpu.SemaphoreType.DMA,
        ],
    )
    def k(x_hbm, src_hbm, dst_hbm, w_hbm, z_hbm, part_hbm,
          src_i, dst_i, w_i, r0, r1, r2, acc,
          gsem0, gsem1, gsem2, ssem0, ssem1, ssem2, isem0, isem1, isem2,
          dsem0, dsem1, dsem2):
        c = lax.axis_index("c")
        s = lax.axis_index("s")
        wid = c * NS + s
        rbuf = (r0, r1, r2)
        gsem = (gsem0, gsem1, gsem2)
        ssem = (ssem0, ssem1, ssem2)
        isem = (isem0, isem1, isem2)
        dsem = (dsem0, dsem1, dsem2)

        def sw_load(j, slot):
            pltpu.async_copy(src_hbm.at[wid, j], src_i.at[slot], isem[slot])
            pltpu.async_copy(w_hbm.at[wid, j], w_i.at[slot], isem[slot])

        def sw_wait(j, slot):
            pltpu.make_async_copy(
                src_hbm.at[wid, j], src_i.at[slot], isem[slot]).wait()
            pltpu.make_async_copy(
                w_hbm.at[wid, j], w_i.at[slot], isem[slot]).wait()

        def dst_load(j, slot):
            pltpu.async_copy(dst_hbm.at[wid, j], dst_i.at[slot], dsem[slot])

        def dst_wait(j, slot):
            pltpu.make_async_copy(
                dst_hbm.at[wid, j], dst_i.at[slot], dsem[slot]).wait()

        def zslice(lo, sz):
            pltpu.sync_copy(z_hbm.at[pl.ds(lo, sz)], acc.at[pl.ds(lo, sz)])

        @pl.when(s < NS - 1)
        def _():
            zslice(s * rpt, rpt)

        @pl.when(s == NS - 1)
        def _():
            zslice((NS - 1) * rpt, last)

        sw_load(0, 0)
        sw_load(1, 1)
        sw_load(2, 2)
        dst_load(0, 0)
        dst_load(1, 1)
        plsc.subcore_barrier()
        sw_wait(0, 0)
        sw_wait(1, 1)
        pltpu.async_copy(x_hbm.at[src_i.at[0]], r0, gsem0)
        pltpu.async_copy(x_hbm.at[src_i.at[1]], r1, gsem1)

        zeros16 = jnp.zeros((L,), jnp.int32)

        def round_(g, carry):
            for b in range(3):
                j = 3 * g + b
                b2 = (b + 2) % 3

                pltpu.make_async_copy(
                    x_hbm.at[src_i.at[b]], rbuf[b], gsem[b]).wait()

                bsplat = zeros16 + b

                @plsc.parallel_loop(0, 128, unroll=8)
                def _(e):
                    esplat = zeros16 + e
                    wv = plsc.load_gather(w_i, [bsplat, esplat])
                    for k8 in range(d // L):
                        sl = pl.ds(k8 * L, L)
                        rbuf[b][e, sl] = rbuf[b][e, sl] * wv

                @pl.when(j > 0)
                def _():
                    pltpu.make_async_copy(
                        rbuf[b2], acc.at[dst_i.at[b2]], ssem[b2]).wait()

                @pl.when(j + 2 < nb)
                def _():
                    dst_load(j + 2, b2)
                    sw_wait(j + 2, b2)
                    pltpu.async_copy(
                        x_hbm.at[src_i.at[b2]], rbuf[b2], gsem[b2])

                dst_wait(j, b)
                pltpu.async_copy(
                    rbuf[b], acc.at[dst_i.at[b]], ssem[b], add=True)

                @pl.when(j + 3 < nb)
                def _():
                    sw_load(j + 3, b)
            return carry

        lax.fori_loop(0, nb // 3, round_, 0)
        blast = (nb - 1) % 3
        pltpu.make_async_copy(
            rbuf[blast], acc.at[dst_i.at[blast]], ssem[blast]).wait()
        plsc.subcore_barrier()

        @pl.when(s < NS - 1)
        def _():
            pltpu.sync_copy(acc.at[pl.ds(s * rpt, rpt)],
                            part_hbm.at[c, pl.ds(s * rpt, rpt)])

        @pl.when(s == NS - 1)
        def _():
            pltpu.sync_copy(acc.at[pl.ds((NS - 1) * rpt, last)],
                            part_hbm.at[c, pl.ds((NS - 1) * rpt, last)])

    return k(x, srcp, dstp, wp, zeros)


def _combine(x, part):
    n, d = x.shape
    blk = 1000

    def body(x_ref, p_ref, o_ref):
        f = p_ref[0] + p_ref[1]
        o_ref[...] = x_ref[...] + jnp.maximum(f, 0.0)

    return pl.pallas_call(
        body,
        grid=(n // blk,),
        in_specs=[
            pl.BlockSpec((blk, d), lambda i: (i, 0)),
            pl.BlockSpec((NC, blk, d), lambda i: (0, i, 0)),
        ],
        out_specs=pl.BlockSpec((blk, d), lambda i: (i, 0)),
        out_shape=jax.ShapeDtypeStruct((n, d), jnp.float32),
    )(x, part)


def kernel(x, edge_index, edge_values):
    n, d = x.shape
    e = edge_values.shape[0]
    nb = -(-e // (NW * 128))
    nb += (-nb) % 3
    ept = nb * 128
    epad = ept * NW
    dst = edge_index[0]
    src = edge_index[1]
    pad = epad - e
    pad_idx = jnp.arange(pad, dtype=dst.dtype) % n
    srcp = jnp.concatenate([src, pad_idx]).reshape(NW, nb, 128)
    dstp = jnp.concatenate([dst, pad_idx]).reshape(NW, nb, 128)
    wp = jnp.pad(edge_values, (0, pad)).reshape(NW, nb, 128)
    zeros = jnp.zeros((n, d), jnp.float32)
    part = _sc_spmm(x, srcp, dstp, wp, zeros, n, d, nb)
    return _combine(x, part)


# speedup vs baseline: 1.0449x; 1.0449x over previous
import functools

import jax
import jax.numpy as jnp
from jax import lax
from jax.experimental import pallas as pl
from jax.experimental.pallas import tpu as pltpu
from jax.experimental.pallas import tpu_sc as plsc

NC = 2
NS = 16
L = 16
NW = NC * NS


def _sc_spmm(x, srcp, dstp, wp, n, d, nb):
    ept = nb * 128
    mesh = plsc.VectorSubcoreMesh(
        core_axis_name="c", subcore_axis_name="s", num_cores=NC, num_subcores=NS
    )
    rpt = 8 * (-(-n // (NS * 8)))
    last = n - (NS - 1) * rpt

    @functools.partial(
        pl.kernel,
        out_type=jax.ShapeDtypeStruct((NC, n, d), jnp.float32),
        mesh=mesh,
        compiler_params=pltpu.CompilerParams(needs_layout_passes=False),
        scratch_types=[
            pltpu.VMEM((3, 128), jnp.int32),
            pltpu.VMEM((3, 128), jnp.int32),
            pltpu.VMEM((3, 128), jnp.float32),
            pltpu.VMEM((128, d), jnp.float32),
            pltpu.VMEM((128, d), jnp.float32),
            pltpu.VMEM((128, d), jnp.float32),
            pltpu.VMEM_SHARED((n, d), jnp.float32),
            pltpu.SemaphoreType.DMA,
            pltpu.SemaphoreType.DMA,
            pltpu.SemaphoreType.DMA,
            pltpu.SemaphoreType.DMA,
            pltpu.SemaphoreType.DMA,
            pltpu.SemaphoreType.DMA,
            pltpu.SemaphoreType.DMA,
            pltpu.SemaphoreType.DMA,
            pltpu.SemaphoreType.DMA,
            pltpu.SemaphoreType.DMA,
            pltpu.SemaphoreType.DMA,
            pltpu.SemaphoreType.DMA,
        ],
    )
    def k(x_hbm, src_hbm, dst_hbm, w_hbm, part_hbm,
          src_i, dst_i, w_i, r0, r1, r2, acc,
          gsem0, gsem1, gsem2, ssem0, ssem1, ssem2, isem0, isem1, isem2,
          dsem0, dsem1, dsem2):
        c = lax.axis_index("c")
        s = lax.axis_index("s")
        wid = c * NS + s
        rbuf = (r0, r1, r2)
        gsem = (gsem0, gsem1, gsem2)
        ssem = (ssem0, ssem1, ssem2)
        isem = (isem0, isem1, isem2)
        dsem = (dsem0, dsem1, dsem2)

        def sw_load(j, slot):
            base = wid * ept + j * 128
            pltpu.async_copy(
                src_hbm.at[pl.ds(base, 128)], src_i.at[slot], isem[slot])
            pltpu.async_copy(
                w_hbm.at[pl.ds(base, 128)], w_i.at[slot], isem[slot])

        def sw_wait(j, slot):
            base = wid * ept + j * 128
            pltpu.make_async_copy(
                src_hbm.at[pl.ds(base, 128)], src_i.at[slot], isem[slot]).wait()
            pltpu.make_async_copy(
                w_hbm.at[pl.ds(base, 128)], w_i.at[slot], isem[slot]).wait()

        def dst_load(j, slot):
            base = wid * ept + j * 128
            pltpu.async_copy(
                dst_hbm.at[pl.ds(base, 128)], dst_i.at[slot], dsem[slot])

        def dst_wait(j, slot):
            base = wid * ept + j * 128
            pltpu.make_async_copy(
                dst_hbm.at[pl.ds(base, 128)], dst_i.at[slot], dsem[slot]).wait()

        fzeros16 = jnp.zeros((L,), jnp.float32)

        def zrow(i, carry):
            for k8 in range(d // L):
                r0[i, pl.ds(k8 * L, L)] = fzeros16
            return carry

        lax.fori_loop(0, 128, zrow, 0)

        def zfill(lo, sz):
            for kk in range(sz // 128):
                pltpu.sync_copy(r0, acc.at[pl.ds(lo + kk * 128, 128)])
            rem = sz % 128
            if rem:
                pltpu.sync_copy(r0.at[pl.ds(0, rem)],
                                acc.at[pl.ds(lo + (sz // 128) * 128, rem)])

        @pl.when(s < NS - 1)
        def _():
            zfill(s * rpt, rpt)

        @pl.when(s == NS - 1)
        def _():
            zfill((NS - 1) * rpt, last)

        sw_load(0, 0)
        sw_load(1, 1)
        sw_load(2, 2)
        dst_load(0, 0)
        dst_load(1, 1)
        plsc.subcore_barrier()
        sw_wait(0, 0)
        sw_wait(1, 1)
        pltpu.async_copy(x_hbm.at[src_i.at[0]], r0, gsem0)
        pltpu.async_copy(x_hbm.at[src_i.at[1]], r1, gsem1)

        zeros16 = jnp.zeros((L,), jnp.int32)

        def round_(g, carry):
            for b in range(3):
                j = 3 * g + b
                b2 = (b + 2) % 3

                pltpu.make_async_copy(
                    x_hbm.at[src_i.at[b]], rbuf[b], gsem[b]).wait()

                bsplat = zeros16 + b

                @plsc.parallel_loop(0, 128, unroll=4)
                def _(e):
                    esplat = zeros16 + e
                    wv = plsc.load_gather(w_i, [bsplat, esplat])
                    for k8 in range(d // L):
                        sl = pl.ds(k8 * L, L)
                        rbuf[b][e, sl] = rbuf[b][e, sl] * wv

                @pl.when(j > 0)
                def _():
                    pltpu.make_async_copy(
                        rbuf[b2], acc.at[dst_i.at[b2]], ssem[b2]).wait()

                @pl.when(j + 2 < nb)
                def _():
                    dst_load(j + 2, b2)
                    sw_wait(j + 2, b2)
                    pltpu.async_copy(
                        x_hbm.at[src_i.at[b2]], rbuf[b2], gsem[b2])

                dst_wait(j, b)
                pltpu.async_copy(
                    rbuf[b], acc.at[dst_i.at[b]], ssem[b], add=True)

                @pl.when(j + 3 < nb)
                def _():
                    sw_load(j + 3, b)
            return carry

        lax.fori_loop(0, nb // 3, round_, 0)
        blast = (nb - 1) % 3
        pltpu.make_async_copy(
            rbuf[blast], acc.at[dst_i.at[blast]], ssem[blast]).wait()
        plsc.subcore_barrier()

        @pl.when(s < NS - 1)
        def _():
            pltpu.sync_copy(acc.at[pl.ds(s * rpt, rpt)],
                            part_hbm.at[c, pl.ds(s * rpt, rpt)])

        @pl.when(s == NS - 1)
        def _():
            pltpu.sync_copy(acc.at[pl.ds((NS - 1) * rpt, last)],
                            part_hbm.at[c, pl.ds((NS - 1) * rpt, last)])

    return k(x, srcp, dstp, wp)


def _combine(x, part):
    n, d = x.shape
    blk = 1000

    def body(x_ref, p_ref, o_ref):
        f = p_ref[0] + p_ref[1]
        o_ref[...] = x_ref[...] + jnp.maximum(f, 0.0)

    return pl.pallas_call(
        body,
        grid=(n // blk,),
        in_specs=[
            pl.BlockSpec((blk, d), lambda i: (i, 0)),
            pl.BlockSpec((NC, blk, d), lambda i: (0, i, 0)),
        ],
        out_specs=pl.BlockSpec((blk, d), lambda i: (i, 0)),
        out_shape=jax.ShapeDtypeStruct((n, d), jnp.float32),
    )(x, part)


def kernel(x, edge_index, edge_values):
    n, d = x.shape
    e = edge_values.shape[0]
    nb = -(-e // (NW * 128))
    nb += (-nb) % 3
    ept = nb * 128
    epad = ept * NW
    dst = edge_index[0]
    src = edge_index[1]
    pad = epad - e
    pad_idx = jnp.arange(pad, dtype=dst.dtype) % n
    srcp = jnp.concatenate([src, pad_idx])
    dstp = jnp.concatenate([dst, pad_idx])
    wp = jnp.pad(edge_values, (0, pad))
    part = _sc_spmm(x, srcp, dstp, wp, n, d, nb)
    return _combine(x, part)


# speedup vs baseline: 1.0581x; 1.0127x over previous
import functools

import jax
import jax.numpy as jnp
from jax import lax
from jax.experimental import pallas as pl
from jax.experimental.pallas import tpu as pltpu
from jax.experimental.pallas import tpu_sc as plsc

NC = 2
NS = 16
L = 16
NW = NC * NS


def _sc_spmm(x, srcp, dstp, wp, n, d, nb):
    ept = nb * 128
    mesh = plsc.VectorSubcoreMesh(
        core_axis_name="c", subcore_axis_name="s", num_cores=NC, num_subcores=NS
    )
    rpt = 8 * (-(-n // (NS * 8)))
    last = n - (NS - 1) * rpt

    @functools.partial(
        pl.kernel,
        out_type=jax.ShapeDtypeStruct((NC, n, d), jnp.float32),
        mesh=mesh,
        compiler_params=pltpu.CompilerParams(needs_layout_passes=False),
        scratch_types=[
            pltpu.VMEM((3, 128), jnp.int32),
            pltpu.VMEM((3, 128), jnp.int32),
            pltpu.VMEM((3, 128), jnp.float32),
            pltpu.VMEM((128, d), jnp.float32),
            pltpu.VMEM((128, d), jnp.float32),
            pltpu.VMEM((128, d), jnp.float32),
            pltpu.VMEM_SHARED((n, d), jnp.float32),
            pltpu.SemaphoreType.DMA,
            pltpu.SemaphoreType.DMA,
            pltpu.SemaphoreType.DMA,
            pltpu.SemaphoreType.DMA,
            pltpu.SemaphoreType.DMA,
            pltpu.SemaphoreType.DMA,
            pltpu.SemaphoreType.DMA,
            pltpu.SemaphoreType.DMA,
            pltpu.SemaphoreType.DMA,
            pltpu.SemaphoreType.DMA,
            pltpu.SemaphoreType.DMA,
            pltpu.SemaphoreType.DMA,
        ],
    )
    def k(x_hbm, src_hbm, dst_hbm, w_hbm, part_hbm,
          src_i, dst_i, w_i, r0, r1, r2, acc,
          gsem0, gsem1, gsem2, ssem0, ssem1, ssem2, isem0, isem1, isem2,
          dsem0, dsem1, dsem2):
        c = lax.axis_index("c")
        s = lax.axis_index("s")
        wid = c * NS + s
        rbuf = (r0, r1, r2)
        gsem = (gsem0, gsem1, gsem2)
        ssem = (ssem0, ssem1, ssem2)
        isem = (isem0, isem1, isem2)
        dsem = (dsem0, dsem1, dsem2)

        def sw_load(j, slot):
            base = wid * ept + j * 128
            pltpu.async_copy(
                src_hbm.at[pl.ds(base, 128)], src_i.at[slot], isem[slot])
            pltpu.async_copy(
                w_hbm.at[pl.ds(base, 128)], w_i.at[slot], isem[slot])

        def sw_wait(j, slot):
            base = wid * ept + j * 128
            pltpu.make_async_copy(
                src_hbm.at[pl.ds(base, 128)], src_i.at[slot], isem[slot]).wait()
            pltpu.make_async_copy(
                w_hbm.at[pl.ds(base, 128)], w_i.at[slot], isem[slot]).wait()

        def dst_load(j, slot):
            base = wid * ept + j * 128
            pltpu.async_copy(
                dst_hbm.at[pl.ds(base, 128)], dst_i.at[slot], dsem[slot])

        def dst_wait(j, slot):
            base = wid * ept + j * 128
            pltpu.make_async_copy(
                dst_hbm.at[pl.ds(base, 128)], dst_i.at[slot], dsem[slot]).wait()

        fzeros16 = jnp.zeros((L,), jnp.float32)

        def zrow(i, carry):
            for k8 in range(d // L):
                r0[i, pl.ds(k8 * L, L)] = fzeros16
            return carry

        lax.fori_loop(0, 128, zrow, 0)

        def zfill(lo, sz):
            for kk in range(sz // 128):
                pltpu.sync_copy(r0, acc.at[pl.ds(lo + kk * 128, 128)])
            rem = sz % 128
            if rem:
                pltpu.sync_copy(r0.at[pl.ds(0, rem)],
                                acc.at[pl.ds(lo + (sz // 128) * 128, rem)])

        @pl.when(s < NS - 1)
        def _():
            zfill(s * rpt, rpt)

        @pl.when(s == NS - 1)
        def _():
            zfill((NS - 1) * rpt, last)

        sw_load(0, 0)
        sw_load(1, 1)
        sw_load(2, 2)
        dst_load(0, 0)
        dst_load(1, 1)
        plsc.subcore_barrier()
        sw_wait(0, 0)
        sw_wait(1, 1)
        pltpu.async_copy(x_hbm.at[src_i.at[0]], r0, gsem0)
        pltpu.async_copy(x_hbm.at[src_i.at[1]], r1, gsem1)

        zeros16 = jnp.zeros((L,), jnp.int32)

        def round_(g, carry):
            for b in range(3):
                j = 3 * g + b
                b2 = (b + 2) % 3

                pltpu.make_async_copy(
                    x_hbm.at[src_i.at[b]], rbuf[b], gsem[b]).wait()

                bsplat = zeros16 + b

                @plsc.parallel_loop(0, 128, unroll=4)
                def _(e):
                    esplat = zeros16 + e
                    wv = plsc.load_gather(w_i, [bsplat, esplat])
                    for k8 in range(d // L):
                        sl = pl.ds(k8 * L, L)
                        rbuf[b][e, sl] = rbuf[b][e, sl] * wv

                @pl.when(j > 0)
                def _():
                    pltpu.make_async_copy(
                        rbuf[b2], acc.at[dst_i.at[b2]], ssem[b2]).wait()

                dst_wait(j, b)
                pltpu.async_copy(
                    rbuf[b], acc.at[dst_i.at[b]], ssem[b], add=True)

                @pl.when(j + 2 < nb)
                def _():
                    dst_load(j + 2, b2)
                    sw_wait(j + 2, b2)
                    pltpu.async_copy(
                        x_hbm.at[src_i.at[b2]], rbuf[b2], gsem[b2])

                @pl.when(j + 3 < nb)
                def _():
                    sw_load(j + 3, b)
            return carry

        lax.fori_loop(0, nb // 3, round_, 0)
        blast = (nb - 1) % 3
        pltpu.make_async_copy(
            rbuf[blast], acc.at[dst_i.at[blast]], ssem[blast]).wait()
        plsc.subcore_barrier()

        @pl.when(s < NS - 1)
        def _():
            pltpu.sync_copy(acc.at[pl.ds(s * rpt, rpt)],
                            part_hbm.at[c, pl.ds(s * rpt, rpt)])

        @pl.when(s == NS - 1)
        def _():
            pltpu.sync_copy(acc.at[pl.ds((NS - 1) * rpt, last)],
                            part_hbm.at[c, pl.ds((NS - 1) * rpt, last)])

    return k(x, srcp, dstp, wp)


def _combine(x, part):
    n, d = x.shape
    blk = 2000

    def body(x_ref, p_ref, o_ref):
        f = p_ref[0] + p_ref[1]
        o_ref[...] = x_ref[...] + jnp.maximum(f, 0.0)

    return pl.pallas_call(
        body,
        grid=(n // blk,),
        in_specs=[
            pl.BlockSpec((blk, d), lambda i: (i, 0)),
            pl.BlockSpec((NC, blk, d), lambda i: (0, i, 0)),
        ],
        out_specs=pl.BlockSpec((blk, d), lambda i: (i, 0)),
        out_shape=jax.ShapeDtypeStruct((n, d), jnp.float32),
    )(x, part)


def kernel(x, edge_index, edge_values):
    n, d = x.shape
    e = edge_values.shape[0]
    nb = -(-e // (NW * 128))
    nb += (-nb) % 3
    ept = nb * 128
    epad = ept * NW
    dst = edge_index[0]
    src = edge_index[1]
    pad = epad - e
    pad_idx = jnp.arange(pad, dtype=dst.dtype) % n
    srcp = jnp.concatenate([src, pad_idx])
    dstp = jnp.concatenate([dst, pad_idx])
    wp = jnp.pad(edge_values, (0, pad))
    part = _sc_spmm(x, srcp, dstp, wp, n, d, nb)
    return _combine(x, part)
